# Initial kernel scaffold; baseline (speedup 1.0000x reference)
import jax
import jax.numpy as jnp
from jax.experimental import pallas as pl


def kernel(x, edge_index, batch, params):
    raise NotImplementedError("write your pallas kernel here")


# speedup vs baseline: 4.2824x; 4.2824x over previous
import functools

import jax
import jax.numpy as jnp
from jax import lax
from jax.experimental import pallas as pl
from jax.experimental.pallas import tpu as pltpu
from jax.experimental.pallas import tpu_sc as plsc

N_NODES = 10000
D = 128
N_CLASSES = 40
N_LAYERS = 4
EPS = 1e-5

NC = 2
NS = 16
NW = NC * NS
LANES = 16

CHUNK = 128
E_PAD = 327680
TRASH = N_NODES

DEG_NCHUNK = E_PAD // (NW * CHUNK)
DEG_ROWS = 10240
DEG_ZROWS = DEG_ROWS // NS

AGG_NCHUNK = E_PAD // (NS * CHUNK)
ACC_ROWS = 6144
ACC_ZCH = ACC_ROWS // NS // CHUNK
OUT_ROWS = 5120
OUT_SUB = OUT_ROWS // NS
NBUF = 2

ROW_BLK = 2000
GRID = N_NODES // ROW_BLK


def _sc_mesh():
    return plsc.VectorSubcoreMesh(
        core_axis_name="c", subcore_axis_name="s",
        num_cores=NC, num_subcores=NS)


def _sc_degree(dst_r, ones, zeros):
    @functools.partial(
        pl.kernel,
        mesh=_sc_mesh(),
        out_type=jax.ShapeDtypeStruct((NC, DEG_ROWS), jnp.float32),
        scratch_types=[
            pltpu.VMEM((DEG_NCHUNK, CHUNK), jnp.int32),
            pltpu.VMEM((CHUNK,), jnp.float32),
            pltpu.VMEM((DEG_ZROWS,), jnp.float32),
            pltpu.VMEM_SHARED((DEG_ROWS,), jnp.float32),
        ],
    )
    def deg_kernel(dst_hbm, ones_hbm, zeros_hbm, out_hbm, dst_v, ones_v, z_v, acc):
        c = lax.axis_index("c")
        s = lax.axis_index("s")
        wid = s * NC + c
        pltpu.sync_copy(dst_hbm.at[wid], dst_v)
        pltpu.sync_copy(ones_hbm, ones_v)
        pltpu.sync_copy(zeros_hbm, z_v)
        pltpu.sync_copy(z_v, acc.at[pl.ds(s * DEG_ZROWS, DEG_ZROWS)])
        plsc.subcore_barrier()

        @pl.loop(0, DEG_NCHUNK)
        def _(j):
            pltpu.sync_copy(ones_v, acc.at[dst_v.at[j]], add=True)

        plsc.subcore_barrier()
        pltpu.sync_copy(acc.at[pl.ds(s * DEG_ZROWS, DEG_ZROWS)],
                        out_hbm.at[c, pl.ds(s * DEG_ZROWS, DEG_ZROWS)])

    return deg_kernel(dst_r, ones, zeros)


def _sc_aggregate(hp, src_r, dst_r, zrows):
    @functools.partial(
        pl.kernel,
        mesh=_sc_mesh(),
        out_type=jax.ShapeDtypeStruct((NC, OUT_ROWS, D), jnp.float32),
        scratch_types=[
            pltpu.VMEM((AGG_NCHUNK, CHUNK), jnp.int32),
            pltpu.VMEM((AGG_NCHUNK, CHUNK), jnp.int32),
            [pltpu.VMEM((CHUNK, D), jnp.float32) for _ in range(NBUF)],
            pltpu.VMEM_SHARED((ACC_ROWS, D), jnp.float32),
            [pltpu.SemaphoreType.DMA for _ in range(NBUF)],
        ],
    )
    def agg_kernel(hp_hbm, src_hbm, dst_hbm, z_hbm, out_hbm,
                   src_v, dst_v, rbufs, acc, sems):
        c = lax.axis_index("c")
        s = lax.axis_index("s")
        pltpu.sync_copy(src_hbm.at[s], src_v)
        pltpu.sync_copy(dst_hbm.at[s], dst_v)

        base = c * OUT_ROWS

        @pl.loop(0, AGG_NCHUNK)
        def _(j):
            for v in range(CHUNK // LANES):
                d = dst_v[j, pl.ds(v * LANES, LANES)]
                dl = d - base
                ok = (dl >= 0) & (dl < OUT_ROWS)
                alt = OUT_ROWS + lax.bitwise_and(d, 63)
                dst_v[j, pl.ds(v * LANES, LANES)] = jnp.where(ok, dl, alt)

        pltpu.sync_copy(z_hbm, rbufs[0])
        for z in range(ACC_ZCH):
            pltpu.sync_copy(
                rbufs[0],
                acc.at[pl.ds(s * (ACC_ZCH * CHUNK) + z * CHUNK, CHUNK)])
        plsc.subcore_barrier()

        @pl.loop(0, AGG_NCHUNK // NBUF)
        def _(i):
            j = i * NBUF
            cps = [
                pltpu.async_copy(hp_hbm.at[src_v.at[j + b]], rbufs[b], sems[b])
                for b in range(NBUF)
            ]
            for b in range(NBUF):
                cps[b].wait()
                pltpu.sync_copy(rbufs[b], acc.at[dst_v.at[j + b]], add=True)

        plsc.subcore_barrier()
        pltpu.sync_copy(acc.at[pl.ds(s * OUT_SUB, OUT_SUB)],
                        out_hbm.at[c, pl.ds(s * OUT_SUB, OUT_SUB)])

    return agg_kernel(hp, src_r, dst_r, zrows)


def _dot(a, w):
    return jnp.dot(a, w, preferred_element_type=jnp.float32,
                   precision=lax.Precision.HIGHEST)


def _first_layer(x, w0, b0, deg0, deg1):
    def body(x_ref, w_ref, b_ref, d0_ref, d1_ref, hp_ref, dinv_ref):
        deg = d0_ref[...] + d1_ref[...] + 1.0
        dinv = lax.rsqrt(jnp.maximum(deg, 1.0))
        dinv_ref[...] = dinv
        hp_ref[...] = (_dot(x_ref[...], w_ref[...]) + b_ref[...]) * dinv

    return pl.pallas_call(
        body,
        grid=(GRID,),
        in_specs=[
            pl.BlockSpec((ROW_BLK, D), lambda i: (i, 0)),
            pl.BlockSpec((D, D), lambda i: (0, 0)),
            pl.BlockSpec((1, D), lambda i: (0, 0)),
            pl.BlockSpec((ROW_BLK, 1), lambda i: (i, 0)),
            pl.BlockSpec((ROW_BLK, 1), lambda i: (i, 0)),
        ],
        out_specs=[
            pl.BlockSpec((ROW_BLK, D), lambda i: (i, 0)),
            pl.BlockSpec((ROW_BLK, 1), lambda i: (i, 0)),
        ],
        out_shape=[
            jax.ShapeDtypeStruct((N_NODES, D), jnp.float32),
            jax.ShapeDtypeStruct((N_NODES, 1), jnp.float32),
        ],
    )(x, w0, b0, deg0, deg1)


def _combine_stats(aggf, hp, dinv):
    def body(a_ref, hp_ref, dinv_ref, t_ref, st_ref):
        t = (a_ref[...] + hp_ref[...]) * dinv_ref[...]
        t_ref[...] = t
        s0 = jnp.sum(t, axis=0, keepdims=True)
        s1 = jnp.sum(t * t, axis=0, keepdims=True)
        blk = jnp.concatenate([s0, s1, jnp.zeros((6, D), jnp.float32)], axis=0)

        @pl.when(pl.program_id(0) == 0)
        def _():
            st_ref[...] = blk

        @pl.when(pl.program_id(0) != 0)
        def _():
            st_ref[...] += blk

    return pl.pallas_call(
        body,
        grid=(GRID,),
        in_specs=[
            pl.BlockSpec((ROW_BLK, D), lambda i: (i, 0)),
            pl.BlockSpec((ROW_BLK, D), lambda i: (i, 0)),
            pl.BlockSpec((ROW_BLK, 1), lambda i: (i, 0)),
        ],
        out_specs=[
            pl.BlockSpec((ROW_BLK, D), lambda i: (i, 0)),
            pl.BlockSpec((8, D), lambda i: (0, 0)),
        ],
        out_shape=[
            jax.ShapeDtypeStruct((N_NODES, D), jnp.float32),
            jax.ShapeDtypeStruct((8, D), jnp.float32),
        ],
    )(aggf, hp, dinv)


def _bn_relu_matmul(t, st, g, be, wn, bn_, dinv):
    def body(t_ref, st_ref, g_ref, be_ref, w_ref, b_ref, dinv_ref, hp_ref):
        n = jnp.float32(N_NODES)
        mean = st_ref[0:1, :] / n
        var = st_ref[1:2, :] / n - mean * mean
        inv = lax.rsqrt(var + EPS)
        xb = jax.nn.relu((t_ref[...] - mean) * inv * g_ref[...] + be_ref[...])
        hp_ref[...] = (_dot(xb, w_ref[...]) + b_ref[...]) * dinv_ref[...]

    return pl.pallas_call(
        body,
        grid=(GRID,),
        in_specs=[
            pl.BlockSpec((ROW_BLK, D), lambda i: (i, 0)),
            pl.BlockSpec((8, D), lambda i: (0, 0)),
            pl.BlockSpec((1, D), lambda i: (0, 0)),
            pl.BlockSpec((1, D), lambda i: (0, 0)),
            pl.BlockSpec((D, D), lambda i: (0, 0)),
            pl.BlockSpec((1, D), lambda i: (0, 0)),
            pl.BlockSpec((ROW_BLK, 1), lambda i: (i, 0)),
        ],
        out_specs=pl.BlockSpec((ROW_BLK, D), lambda i: (i, 0)),
        out_shape=jax.ShapeDtypeStruct((N_NODES, D), jnp.float32),
    )(t, st, g, be, wn, bn_, dinv)


def _bn_relu_head(t, st, g, be, w1, b1, w2, b2):
    def body(t_ref, st_ref, g_ref, be_ref, w1_ref, b1_ref, w2_ref, b2_ref, o_ref):
        n = jnp.float32(N_NODES)
        mean = st_ref[0:1, :] / n
        var = st_ref[1:2, :] / n - mean * mean
        inv = lax.rsqrt(var + EPS)
        xb = jax.nn.relu((t_ref[...] - mean) * inv * g_ref[...] + be_ref[...])
        h1 = jax.nn.relu(_dot(xb, w1_ref[...]) + b1_ref[...])
        o_ref[...] = _dot(h1, w2_ref[...]) + b2_ref[...]

    return pl.pallas_call(
        body,
        grid=(GRID,),
        in_specs=[
            pl.BlockSpec((ROW_BLK, D), lambda i: (i, 0)),
            pl.BlockSpec((8, D), lambda i: (0, 0)),
            pl.BlockSpec((1, D), lambda i: (0, 0)),
            pl.BlockSpec((1, D), lambda i: (0, 0)),
            pl.BlockSpec((D, D), lambda i: (0, 0)),
            pl.BlockSpec((1, D), lambda i: (0, 0)),
            pl.BlockSpec((D, D), lambda i: (0, 0)),
            pl.BlockSpec((1, D), lambda i: (0, 0)),
        ],
        out_specs=pl.BlockSpec((ROW_BLK, D), lambda i: (i, 0)),
        out_shape=jax.ShapeDtypeStruct((N_NODES, D), jnp.float32),
    )(t, st, g, be, w1, b1, w2, b2)


def kernel(x, edge_index, batch, params):
    del batch

    src = edge_index[0].astype(jnp.int32)
    dst = edge_index[1].astype(jnp.int32)
    e = src.shape[0]
    pad = E_PAD - e
    src_f = jnp.concatenate([src, jnp.zeros((pad,), jnp.int32)])
    dst_f = jnp.concatenate([dst, jnp.full((pad,), TRASH, jnp.int32)])
    src_a = src_f.reshape(NS, AGG_NCHUNK, CHUNK)
    dst_a = dst_f.reshape(NS, AGG_NCHUNK, CHUNK)
    dst_d = dst_f.reshape(NW, DEG_NCHUNK, CHUNK)

    ones = jnp.ones((CHUNK,), jnp.float32)
    zeros = jnp.zeros((DEG_ZROWS,), jnp.float32)
    zrows = jnp.zeros((CHUNK, D), jnp.float32)

    deg_p = _sc_degree(dst_d, ones, zeros)
    deg0 = deg_p[0, :N_NODES].reshape(N_NODES, 1)
    deg1 = deg_p[1, :N_NODES].reshape(N_NODES, 1)

    hp, dinv = _first_layer(
        x, params["W0"], params["b0"].reshape(1, D), deg0, deg1)

    for i in range(N_LAYERS):
        agg = _sc_aggregate(hp, src_a, dst_a, zrows)
        aggf = agg.reshape(NC * OUT_ROWS, D)
        t, st = _combine_stats(aggf, hp, dinv)
        g = params[f"g{i}"].reshape(1, D)
        be = params[f"be{i}"].reshape(1, D)
        if i + 1 < N_LAYERS:
            hp = _bn_relu_matmul(
                t, st, g, be,
                params[f"W{i + 1}"], params[f"b{i + 1}"].reshape(1, D), dinv)
        else:
            w2 = jnp.pad(params["hW2"], ((0, 0), (0, D - N_CLASSES)))
            b2 = jnp.pad(params["hb2"], (0, D - N_CLASSES)).reshape(1, D)
            out = _bn_relu_head(
                t, st, g, be,
                params["hW1"], params["hb1"].reshape(1, D), w2, b2)
    return out[:, :N_CLASSES]


# speedup vs baseline: 4.3234x; 1.0096x over previous
import functools

import jax
import jax.numpy as jnp
from jax import lax
from jax.experimental import pallas as pl
from jax.experimental.pallas import tpu as pltpu
from jax.experimental.pallas import tpu_sc as plsc

N_NODES = 10000
D = 128
N_CLASSES = 40
N_LAYERS = 4
EPS = 1e-5

NC = 2
NS = 16
NW = NC * NS
LANES = 16

CHUNK = 128
E_PAD = 327680
TRASH = N_NODES

DEG_NCHUNK = E_PAD // (NW * CHUNK)
DEG_ROWS = 10240
DEG_ZROWS = DEG_ROWS // NS

AGG_NCHUNK = E_PAD // (NS * CHUNK)
ACC_ROWS = 6144
ACC_ZCH = ACC_ROWS // NS // CHUNK
OUT_ROWS = 5120
OUT_SUB = OUT_ROWS // NS
NBUF = 2

ROW_BLK = 2000
GRID = N_NODES // ROW_BLK


def _sc_mesh():
    return plsc.VectorSubcoreMesh(
        core_axis_name="c", subcore_axis_name="s",
        num_cores=NC, num_subcores=NS)


def _sc_degree(dst_r, ones, zeros):
    @functools.partial(
        pl.kernel,
        mesh=_sc_mesh(),
        out_type=jax.ShapeDtypeStruct((NC, DEG_ROWS), jnp.float32),
        scratch_types=[
            pltpu.VMEM((DEG_NCHUNK, CHUNK), jnp.int32),
            pltpu.VMEM((CHUNK,), jnp.float32),
            pltpu.VMEM((DEG_ZROWS,), jnp.float32),
            pltpu.VMEM_SHARED((DEG_ROWS,), jnp.float32),
        ],
    )
    def deg_kernel(dst_hbm, ones_hbm, zeros_hbm, out_hbm, dst_v, ones_v, z_v, acc):
        c = lax.axis_index("c")
        s = lax.axis_index("s")
        wid = s * NC + c
        pltpu.sync_copy(dst_hbm.at[wid], dst_v)
        pltpu.sync_copy(ones_hbm, ones_v)
        pltpu.sync_copy(zeros_hbm, z_v)
        pltpu.sync_copy(z_v, acc.at[pl.ds(s * DEG_ZROWS, DEG_ZROWS)])
        plsc.subcore_barrier()

        @pl.loop(0, DEG_NCHUNK)
        def _(j):
            pltpu.sync_copy(ones_v, acc.at[dst_v.at[j]], add=True)

        plsc.subcore_barrier()
        pltpu.sync_copy(acc.at[pl.ds(s * DEG_ZROWS, DEG_ZROWS)],
                        out_hbm.at[c, pl.ds(s * DEG_ZROWS, DEG_ZROWS)])

    return deg_kernel(dst_r, ones, zeros)


def _sc_aggregate(hp, src_r, dst_r, zrows):
    @functools.partial(
        pl.kernel,
        mesh=_sc_mesh(),
        out_type=jax.ShapeDtypeStruct((NC, OUT_ROWS, D), jnp.float32),
        scratch_types=[
            pltpu.VMEM((AGG_NCHUNK, CHUNK), jnp.int32),
            pltpu.VMEM((AGG_NCHUNK, CHUNK), jnp.int32),
            [pltpu.VMEM((CHUNK, D), jnp.float32) for _ in range(NBUF)],
            pltpu.VMEM_SHARED((ACC_ROWS, D), jnp.float32),
            [pltpu.SemaphoreType.DMA for _ in range(NBUF)],
            [pltpu.SemaphoreType.DMA for _ in range(NBUF)],
        ],
    )
    def agg_kernel(hp_hbm, src_hbm, dst_hbm, z_hbm, out_hbm,
                   src_v, dst_v, rbufs, acc, sems, ssems):
        c = lax.axis_index("c")
        s = lax.axis_index("s")
        pltpu.sync_copy(src_hbm.at[s], src_v)
        pltpu.sync_copy(dst_hbm.at[s], dst_v)

        base = c * OUT_ROWS

        @pl.loop(0, AGG_NCHUNK)
        def _(j):
            for v in range(CHUNK // LANES):
                d = dst_v[j, pl.ds(v * LANES, LANES)]
                dl = d - base
                ok = (dl >= 0) & (dl < OUT_ROWS)
                alt = OUT_ROWS + lax.bitwise_and(d, 63)
                dst_v[j, pl.ds(v * LANES, LANES)] = jnp.where(ok, dl, alt)

        pltpu.sync_copy(z_hbm, rbufs[0])
        for z in range(ACC_ZCH):
            pltpu.sync_copy(
                rbufs[0],
                acc.at[pl.ds(s * (ACC_ZCH * CHUNK) + z * CHUNK, CHUNK)])
        plsc.subcore_barrier()

        def gat(j, b):
            return pltpu.make_async_copy(
                hp_hbm.at[src_v.at[j]], rbufs[b], sems[b])

        def sct(j, b):
            return pltpu.make_async_copy(
                rbufs[b], acc.at[dst_v.at[j]], ssems[b])

        for b in range(NBUF):
            gat(b, b).start()

        rounds = AGG_NCHUNK // NBUF

        @pl.loop(0, rounds - 1)
        def _(i):
            j = i * NBUF
            for b in range(NBUF):
                gat(j + b, b).wait()
                sct(j + b, b).start(add=True)
            for b in range(NBUF):
                sct(j + b, b).wait()
                gat(j + NBUF + b, b).start()

        je = (rounds - 1) * NBUF
        for b in range(NBUF):
            gat(je + b, b).wait()
            sct(je + b, b).start(add=True)
        for b in range(NBUF):
            sct(je + b, b).wait()
        plsc.subcore_barrier()
        pltpu.sync_copy(acc.at[pl.ds(s * OUT_SUB, OUT_SUB)],
                        out_hbm.at[c, pl.ds(s * OUT_SUB, OUT_SUB)])

    return agg_kernel(hp, src_r, dst_r, zrows)


def _dot(a, w):
    return jnp.dot(a, w, preferred_element_type=jnp.float32,
                   precision=lax.Precision.HIGHEST)


def _first_layer(x, w0, b0, deg0, deg1):
    def body(x_ref, w_ref, b_ref, d0_ref, d1_ref, hp_ref, dinv_ref):
        deg = d0_ref[...] + d1_ref[...] + 1.0
        dinv = lax.rsqrt(jnp.maximum(deg, 1.0))
        dinv_ref[...] = dinv
        hp_ref[...] = (_dot(x_ref[...], w_ref[...]) + b_ref[...]) * dinv

    return pl.pallas_call(
        body,
        grid=(GRID,),
        in_specs=[
            pl.BlockSpec((ROW_BLK, D), lambda i: (i, 0)),
            pl.BlockSpec((D, D), lambda i: (0, 0)),
            pl.BlockSpec((1, D), lambda i: (0, 0)),
            pl.BlockSpec((ROW_BLK, 1), lambda i: (i, 0)),
            pl.BlockSpec((ROW_BLK, 1), lambda i: (i, 0)),
        ],
        out_specs=[
            pl.BlockSpec((ROW_BLK, D), lambda i: (i, 0)),
            pl.BlockSpec((ROW_BLK, 1), lambda i: (i, 0)),
        ],
        out_shape=[
            jax.ShapeDtypeStruct((N_NODES, D), jnp.float32),
            jax.ShapeDtypeStruct((N_NODES, 1), jnp.float32),
        ],
    )(x, w0, b0, deg0, deg1)


def _combine_stats(aggf, hp, dinv):
    def body(a_ref, hp_ref, dinv_ref, t_ref, st_ref):
        t = (a_ref[...] + hp_ref[...]) * dinv_ref[...]
        t_ref[...] = t
        s0 = jnp.sum(t, axis=0, keepdims=True)
        s1 = jnp.sum(t * t, axis=0, keepdims=True)
        blk = jnp.concatenate([s0, s1, jnp.zeros((6, D), jnp.float32)], axis=0)

        @pl.when(pl.program_id(0) == 0)
        def _():
            st_ref[...] = blk

        @pl.when(pl.program_id(0) != 0)
        def _():
            st_ref[...] += blk

    return pl.pallas_call(
        body,
        grid=(GRID,),
        in_specs=[
            pl.BlockSpec((ROW_BLK, D), lambda i: (i, 0)),
            pl.BlockSpec((ROW_BLK, D), lambda i: (i, 0)),
            pl.BlockSpec((ROW_BLK, 1), lambda i: (i, 0)),
        ],
        out_specs=[
            pl.BlockSpec((ROW_BLK, D), lambda i: (i, 0)),
            pl.BlockSpec((8, D), lambda i: (0, 0)),
        ],
        out_shape=[
            jax.ShapeDtypeStruct((N_NODES, D), jnp.float32),
            jax.ShapeDtypeStruct((8, D), jnp.float32),
        ],
    )(aggf, hp, dinv)


def _bn_relu_matmul(t, st, g, be, wn, bn_, dinv):
    def body(t_ref, st_ref, g_ref, be_ref, w_ref, b_ref, dinv_ref, hp_ref):
        n = jnp.float32(N_NODES)
        mean = st_ref[0:1, :] / n
        var = st_ref[1:2, :] / n - mean * mean
        inv = lax.rsqrt(var + EPS)
        xb = jax.nn.relu((t_ref[...] - mean) * inv * g_ref[...] + be_ref[...])
        hp_ref[...] = (_dot(xb, w_ref[...]) + b_ref[...]) * dinv_ref[...]

    return pl.pallas_call(
        body,
        grid=(GRID,),
        in_specs=[
            pl.BlockSpec((ROW_BLK, D), lambda i: (i, 0)),
            pl.BlockSpec((8, D), lambda i: (0, 0)),
            pl.BlockSpec((1, D), lambda i: (0, 0)),
            pl.BlockSpec((1, D), lambda i: (0, 0)),
            pl.BlockSpec((D, D), lambda i: (0, 0)),
            pl.BlockSpec((1, D), lambda i: (0, 0)),
            pl.BlockSpec((ROW_BLK, 1), lambda i: (i, 0)),
        ],
        out_specs=pl.BlockSpec((ROW_BLK, D), lambda i: (i, 0)),
        out_shape=jax.ShapeDtypeStruct((N_NODES, D), jnp.float32),
    )(t, st, g, be, wn, bn_, dinv)


def _bn_relu_head(t, st, g, be, w1, b1, w2, b2):
    def body(t_ref, st_ref, g_ref, be_ref, w1_ref, b1_ref, w2_ref, b2_ref, o_ref):
        n = jnp.float32(N_NODES)
        mean = st_ref[0:1, :] / n
        var = st_ref[1:2, :] / n - mean * mean
        inv = lax.rsqrt(var + EPS)
        xb = jax.nn.relu((t_ref[...] - mean) * inv * g_ref[...] + be_ref[...])
        h1 = jax.nn.relu(_dot(xb, w1_ref[...]) + b1_ref[...])
        o_ref[...] = _dot(h1, w2_ref[...]) + b2_ref[...]

    return pl.pallas_call(
        body,
        grid=(GRID,),
        in_specs=[
            pl.BlockSpec((ROW_BLK, D), lambda i: (i, 0)),
            pl.BlockSpec((8, D), lambda i: (0, 0)),
            pl.BlockSpec((1, D), lambda i: (0, 0)),
            pl.BlockSpec((1, D), lambda i: (0, 0)),
            pl.BlockSpec((D, D), lambda i: (0, 0)),
            pl.BlockSpec((1, D), lambda i: (0, 0)),
            pl.BlockSpec((D, D), lambda i: (0, 0)),
            pl.BlockSpec((1, D), lambda i: (0, 0)),
        ],
        out_specs=pl.BlockSpec((ROW_BLK, D), lambda i: (i, 0)),
        out_shape=jax.ShapeDtypeStruct((N_NODES, D), jnp.float32),
    )(t, st, g, be, w1, b1, w2, b2)


def kernel(x, edge_index, batch, params):
    del batch

    src = edge_index[0].astype(jnp.int32)
    dst = edge_index[1].astype(jnp.int32)
    e = src.shape[0]
    pad = E_PAD - e
    src_f = jnp.concatenate([src, jnp.zeros((pad,), jnp.int32)])
    dst_f = jnp.concatenate([dst, jnp.full((pad,), TRASH, jnp.int32)])
    src_a = src_f.reshape(NS, AGG_NCHUNK, CHUNK)
    dst_a = dst_f.reshape(NS, AGG_NCHUNK, CHUNK)
    dst_d = dst_f.reshape(NW, DEG_NCHUNK, CHUNK)

    ones = jnp.ones((CHUNK,), jnp.float32)
    zeros = jnp.zeros((DEG_ZROWS,), jnp.float32)
    zrows = jnp.zeros((CHUNK, D), jnp.float32)

    deg_p = _sc_degree(dst_d, ones, zeros)
    deg0 = deg_p[0, :N_NODES].reshape(N_NODES, 1)
    deg1 = deg_p[1, :N_NODES].reshape(N_NODES, 1)

    hp, dinv = _first_layer(
        x, params["W0"], params["b0"].reshape(1, D), deg0, deg1)

    for i in range(N_LAYERS):
        agg = _sc_aggregate(hp, src_a, dst_a, zrows)
        aggf = agg.reshape(NC * OUT_ROWS, D)
        t, st = _combine_stats(aggf, hp, dinv)
        g = params[f"g{i}"].reshape(1, D)
        be = params[f"be{i}"].reshape(1, D)
        if i + 1 < N_LAYERS:
            hp = _bn_relu_matmul(
                t, st, g, be,
                params[f"W{i + 1}"], params[f"b{i + 1}"].reshape(1, D), dinv)
        else:
            w2 = jnp.pad(params["hW2"], ((0, 0), (0, D - N_CLASSES)))
            b2 = jnp.pad(params["hb2"], (0, D - N_CLASSES)).reshape(1, D)
            out = _bn_relu_head(
                t, st, g, be,
                params["hW1"], params["hb1"].reshape(1, D), w2, b2)
    return out[:, :N_CLASSES]


# speedup vs baseline: 5.6797x; 1.3137x over previous
import functools

import jax
import jax.numpy as jnp
from jax import lax
from jax.experimental import pallas as pl
from jax.experimental.pallas import tpu as pltpu
from jax.experimental.pallas import tpu_sc as plsc

N_NODES = 10000
D = 128
N_CLASSES = 40
N_LAYERS = 4
EPS = 1e-5

NC = 2
NS = 16
NW = NC * NS
LANES = 16

CHUNK = 128
E_PAD = 327680
TRASH = N_NODES

DEG_NCHUNK = E_PAD // (NW * CHUNK)
DEG_ROWS = 10240
DEG_ZROWS = DEG_ROWS // NS

AGG_NCHUNK = E_PAD // (NS * CHUNK)
ACC_ROWS = 6144
ACC_ZCH = ACC_ROWS // NS // CHUNK
OUT_ROWS = 5120
OUT_SUB = OUT_ROWS // NS
NBUF = 2

ROW_BLK = 2000
GRID = N_NODES // ROW_BLK


def _sc_mesh():
    return plsc.VectorSubcoreMesh(
        core_axis_name="c", subcore_axis_name="s",
        num_cores=NC, num_subcores=NS)


def _sc_degree(dst_r, ones, zeros):
    @functools.partial(
        pl.kernel,
        mesh=_sc_mesh(),
        out_type=jax.ShapeDtypeStruct((NC, DEG_ROWS), jnp.float32),
        scratch_types=[
            pltpu.VMEM((DEG_NCHUNK, CHUNK), jnp.int32),
            pltpu.VMEM((CHUNK,), jnp.float32),
            pltpu.VMEM((DEG_ZROWS,), jnp.float32),
            pltpu.VMEM_SHARED((DEG_ROWS,), jnp.float32),
        ],
    )
    def deg_kernel(dst_hbm, ones_hbm, zeros_hbm, out_hbm, dst_v, ones_v, z_v, acc):
        c = lax.axis_index("c")
        s = lax.axis_index("s")
        wid = s * NC + c
        pltpu.sync_copy(dst_hbm.at[wid], dst_v)
        pltpu.sync_copy(ones_hbm, ones_v)
        pltpu.sync_copy(zeros_hbm, z_v)
        pltpu.sync_copy(z_v, acc.at[pl.ds(s * DEG_ZROWS, DEG_ZROWS)])
        plsc.subcore_barrier()

        @pl.loop(0, DEG_NCHUNK)
        def _(j):
            pltpu.sync_copy(ones_v, acc.at[dst_v.at[j]], add=True)

        plsc.subcore_barrier()
        pltpu.sync_copy(acc.at[pl.ds(s * DEG_ZROWS, DEG_ZROWS)],
                        out_hbm.at[c, pl.ds(s * DEG_ZROWS, DEG_ZROWS)])

    return deg_kernel(dst_r, ones, zeros)


BIN_SLOTS = AGG_NCHUNK * CHUNK + CHUNK
BIN_CH = BIN_SLOTS // CHUNK


def _sc_bin(src_r, dst_r):
    @functools.partial(
        pl.kernel,
        mesh=_sc_mesh(),
        out_type=[
            jax.ShapeDtypeStruct((NC, NS, BIN_SLOTS), jnp.int32),
            jax.ShapeDtypeStruct((NC, NS, BIN_SLOTS), jnp.int32),
            jax.ShapeDtypeStruct((NC, NS, 16), jnp.int32),
        ],
        scratch_types=[
            pltpu.VMEM((AGG_NCHUNK, CHUNK), jnp.int32),
            pltpu.VMEM((AGG_NCHUNK, CHUNK), jnp.int32),
            pltpu.VMEM((BIN_SLOTS,), jnp.int32),
            pltpu.VMEM((BIN_SLOTS,), jnp.int32),
            pltpu.VMEM((16,), jnp.int32),
        ],
    )
    def bin_kernel(src_hbm, dst_hbm, srcb_hbm, dstb_hbm, cnt_hbm,
                   src_v, dst_v, sb, db, cb):
        c = lax.axis_index("c")
        s = lax.axis_index("s")
        pltpu.sync_copy(src_hbm.at[s], src_v)
        pltpu.sync_copy(dst_hbm.at[s], dst_v)
        base = c * OUT_ROWS
        lane = lax.iota(jnp.int32, LANES)

        @pl.loop(0, AGG_NCHUNK, init_carry=jnp.int32(0))
        def compact(j, off):
            for v in range(CHUNK // LANES):
                sl = pl.ds(v * LANES, LANES)
                d = dst_v[j, sl]
                sr = src_v[j, sl]
                dl = d - base
                m = (dl >= 0) & (dl < OUT_ROWS)
                w = jnp.where(m, jnp.int32(1), jnp.int32(0))
                sr_c = sr
                dl_c = dl
                cnt = jnp.int32(0)
                for l in range(LANES):
                    wl = w[l]
                    cond = jnp.where(lane == cnt, wl, jnp.int32(0)) > 0
                    sr_c = jnp.where(cond, sr[l], sr_c)
                    dl_c = jnp.where(cond, dl[l], dl_c)
                    cnt = cnt + wl
                sb[pl.ds(off, LANES)] = sr_c
                db[pl.ds(off, LANES)] = dl_c
                off = off + cnt
            return off

        off = compact
        zs = jnp.zeros((LANES,), jnp.int32)
        for t in range(CHUNK // LANES):
            sb[pl.ds(off + t * LANES, LANES)] = zs
            db[pl.ds(off + t * LANES, LANES)] = OUT_ROWS + lane
        pc = lax.bitwise_and(off + (CHUNK - 1), jnp.int32(-CHUNK))
        cb[...] = jnp.zeros((16,), jnp.int32) + pc
        pltpu.sync_copy(sb, srcb_hbm.at[c, s])
        pltpu.sync_copy(db, dstb_hbm.at[c, s])
        pltpu.sync_copy(cb, cnt_hbm.at[c, s])

    return bin_kernel(src_r, dst_r)


def _sc_aggregate(hp, srcb, dstb, cnt, zrows):
    @functools.partial(
        pl.kernel,
        mesh=_sc_mesh(),
        out_type=jax.ShapeDtypeStruct((NC, OUT_ROWS, D), jnp.float32),
        scratch_types=[
            pltpu.VMEM((BIN_CH, CHUNK), jnp.int32),
            pltpu.VMEM((BIN_CH, CHUNK), jnp.int32),
            pltpu.VMEM((16,), jnp.int32),
            [pltpu.VMEM((CHUNK, D), jnp.float32) for _ in range(NBUF)],
            pltpu.VMEM_SHARED((ACC_ROWS, D), jnp.float32),
            [pltpu.SemaphoreType.DMA for _ in range(NBUF)],
            [pltpu.SemaphoreType.DMA for _ in range(NBUF)],
        ],
    )
    def agg_kernel(hp_hbm, src_hbm, dst_hbm, cnt_hbm, z_hbm, out_hbm,
                   src_v, dst_v, cb, rbufs, acc, sems, ssems):
        c = lax.axis_index("c")
        s = lax.axis_index("s")
        pltpu.sync_copy(src_hbm.at[c, s], src_v)
        pltpu.sync_copy(dst_hbm.at[c, s], dst_v)
        pltpu.sync_copy(cnt_hbm.at[c, s], cb)
        nch = cb[...][0] // CHUNK

        pltpu.sync_copy(z_hbm, rbufs[0])
        for z in range(ACC_ZCH):
            pltpu.sync_copy(
                rbufs[0],
                acc.at[pl.ds(s * (ACC_ZCH * CHUNK) + z * CHUNK, CHUNK)])
        plsc.subcore_barrier()

        def gat(j, b):
            return pltpu.make_async_copy(
                hp_hbm.at[src_v.at[j]], rbufs[b], sems[b])

        def sct(j, b):
            return pltpu.make_async_copy(
                rbufs[b], acc.at[dst_v.at[j]], ssems[b])

        for b in range(NBUF):
            @pl.when(b < nch)
            def _(b=b):
                gat(b, b).start()

        @pl.loop(0, AGG_NCHUNK // NBUF)
        def _(i):
            j = i * NBUF
            for b in range(NBUF):
                @pl.when(j + b < nch)
                def _(b=b):
                    gat(j + b, b).wait()
                    sct(j + b, b).start(add=True)
            for b in range(NBUF):
                @pl.when(j + b < nch)
                def _(b=b):
                    sct(j + b, b).wait()

                @pl.when(j + NBUF + b < nch)
                def _(b=b):
                    gat(j + NBUF + b, b).start()

        plsc.subcore_barrier()
        pltpu.sync_copy(acc.at[pl.ds(s * OUT_SUB, OUT_SUB)],
                        out_hbm.at[c, pl.ds(s * OUT_SUB, OUT_SUB)])

    return agg_kernel(hp, srcb, dstb, cnt, zrows)


def _dot(a, w):
    return jnp.dot(a, w, preferred_element_type=jnp.float32,
                   precision=lax.Precision.HIGHEST)


def _first_layer(x, w0, b0, deg0, deg1):
    def body(x_ref, w_ref, b_ref, d0_ref, d1_ref, hp_ref, dinv_ref):
        deg = d0_ref[...] + d1_ref[...] + 1.0
        dinv = lax.rsqrt(jnp.maximum(deg, 1.0))
        dinv_ref[...] = dinv
        hp_ref[...] = (_dot(x_ref[...], w_ref[...]) + b_ref[...]) * dinv

    return pl.pallas_call(
        body,
        grid=(GRID,),
        in_specs=[
            pl.BlockSpec((ROW_BLK, D), lambda i: (i, 0)),
            pl.BlockSpec((D, D), lambda i: (0, 0)),
            pl.BlockSpec((1, D), lambda i: (0, 0)),
            pl.BlockSpec((ROW_BLK, 1), lambda i: (i, 0)),
            pl.BlockSpec((ROW_BLK, 1), lambda i: (i, 0)),
        ],
        out_specs=[
            pl.BlockSpec((ROW_BLK, D), lambda i: (i, 0)),
            pl.BlockSpec((ROW_BLK, 1), lambda i: (i, 0)),
        ],
        out_shape=[
            jax.ShapeDtypeStruct((N_NODES, D), jnp.float32),
            jax.ShapeDtypeStruct((N_NODES, 1), jnp.float32),
        ],
    )(x, w0, b0, deg0, deg1)


def _combine_stats(aggf, hp, dinv):
    def body(a_ref, hp_ref, dinv_ref, t_ref, st_ref):
        t = (a_ref[...] + hp_ref[...]) * dinv_ref[...]
        t_ref[...] = t
        s0 = jnp.sum(t, axis=0, keepdims=True)
        s1 = jnp.sum(t * t, axis=0, keepdims=True)
        blk = jnp.concatenate([s0, s1, jnp.zeros((6, D), jnp.float32)], axis=0)

        @pl.when(pl.program_id(0) == 0)
        def _():
            st_ref[...] = blk

        @pl.when(pl.program_id(0) != 0)
        def _():
            st_ref[...] += blk

    return pl.pallas_call(
        body,
        grid=(GRID,),
        in_specs=[
            pl.BlockSpec((ROW_BLK, D), lambda i: (i, 0)),
            pl.BlockSpec((ROW_BLK, D), lambda i: (i, 0)),
            pl.BlockSpec((ROW_BLK, 1), lambda i: (i, 0)),
        ],
        out_specs=[
            pl.BlockSpec((ROW_BLK, D), lambda i: (i, 0)),
            pl.BlockSpec((8, D), lambda i: (0, 0)),
        ],
        out_shape=[
            jax.ShapeDtypeStruct((N_NODES, D), jnp.float32),
            jax.ShapeDtypeStruct((8, D), jnp.float32),
        ],
    )(aggf, hp, dinv)


def _bn_relu_matmul(t, st, g, be, wn, bn_, dinv):
    def body(t_ref, st_ref, g_ref, be_ref, w_ref, b_ref, dinv_ref, hp_ref):
        n = jnp.float32(N_NODES)
        mean = st_ref[0:1, :] / n
        var = st_ref[1:2, :] / n - mean * mean
        inv = lax.rsqrt(var + EPS)
        xb = jax.nn.relu((t_ref[...] - mean) * inv * g_ref[...] + be_ref[...])
        hp_ref[...] = (_dot(xb, w_ref[...]) + b_ref[...]) * dinv_ref[...]

    return pl.pallas_call(
        body,
        grid=(GRID,),
        in_specs=[
            pl.BlockSpec((ROW_BLK, D), lambda i: (i, 0)),
            pl.BlockSpec((8, D), lambda i: (0, 0)),
            pl.BlockSpec((1, D), lambda i: (0, 0)),
            pl.BlockSpec((1, D), lambda i: (0, 0)),
            pl.BlockSpec((D, D), lambda i: (0, 0)),
            pl.BlockSpec((1, D), lambda i: (0, 0)),
            pl.BlockSpec((ROW_BLK, 1), lambda i: (i, 0)),
        ],
        out_specs=pl.BlockSpec((ROW_BLK, D), lambda i: (i, 0)),
        out_shape=jax.ShapeDtypeStruct((N_NODES, D), jnp.float32),
    )(t, st, g, be, wn, bn_, dinv)


def _bn_relu_head(t, st, g, be, w1, b1, w2, b2):
    def body(t_ref, st_ref, g_ref, be_ref, w1_ref, b1_ref, w2_ref, b2_ref, o_ref):
        n = jnp.float32(N_NODES)
        mean = st_ref[0:1, :] / n
        var = st_ref[1:2, :] / n - mean * mean
        inv = lax.rsqrt(var + EPS)
        xb = jax.nn.relu((t_ref[...] - mean) * inv * g_ref[...] + be_ref[...])
        h1 = jax.nn.relu(_dot(xb, w1_ref[...]) + b1_ref[...])
        o_ref[...] = _dot(h1, w2_ref[...]) + b2_ref[...]

    return pl.pallas_call(
        body,
        grid=(GRID,),
        in_specs=[
            pl.BlockSpec((ROW_BLK, D), lambda i: (i, 0)),
            pl.BlockSpec((8, D), lambda i: (0, 0)),
            pl.BlockSpec((1, D), lambda i: (0, 0)),
            pl.BlockSpec((1, D), lambda i: (0, 0)),
            pl.BlockSpec((D, D), lambda i: (0, 0)),
            pl.BlockSpec((1, D), lambda i: (0, 0)),
            pl.BlockSpec((D, D), lambda i: (0, 0)),
            pl.BlockSpec((1, D), lambda i: (0, 0)),
        ],
        out_specs=pl.BlockSpec((ROW_BLK, D), lambda i: (i, 0)),
        out_shape=jax.ShapeDtypeStruct((N_NODES, D), jnp.float32),
    )(t, st, g, be, w1, b1, w2, b2)


def kernel(x, edge_index, batch, params):
    del batch

    src = edge_index[0].astype(jnp.int32)
    dst = edge_index[1].astype(jnp.int32)
    e = src.shape[0]
    pad = E_PAD - e
    src_f = jnp.concatenate([src, jnp.zeros((pad,), jnp.int32)])
    dst_f = jnp.concatenate([dst, jnp.full((pad,), TRASH, jnp.int32)])
    src_a = src_f.reshape(NS, AGG_NCHUNK, CHUNK)
    dst_a = dst_f.reshape(NS, AGG_NCHUNK, CHUNK)
    dst_d = dst_f.reshape(NW, DEG_NCHUNK, CHUNK)

    ones = jnp.ones((CHUNK,), jnp.float32)
    zeros = jnp.zeros((DEG_ZROWS,), jnp.float32)
    zrows = jnp.zeros((CHUNK, D), jnp.float32)

    srcb, dstb, cntb = _sc_bin(src_a, dst_a)
    srcb4 = srcb.reshape(NC, NS, BIN_CH, CHUNK)
    dstb4 = dstb.reshape(NC, NS, BIN_CH, CHUNK)

    deg_p = _sc_degree(dst_d, ones, zeros)
    deg0 = deg_p[0, :N_NODES].reshape(N_NODES, 1)
    deg1 = deg_p[1, :N_NODES].reshape(N_NODES, 1)

    hp, dinv = _first_layer(
        x, params["W0"], params["b0"].reshape(1, D), deg0, deg1)

    for i in range(N_LAYERS):
        agg = _sc_aggregate(hp, srcb4, dstb4, cntb, zrows)
        aggf = agg.reshape(NC * OUT_ROWS, D)
        t, st = _combine_stats(aggf, hp, dinv)
        g = params[f"g{i}"].reshape(1, D)
        be = params[f"be{i}"].reshape(1, D)
        if i + 1 < N_LAYERS:
            hp = _bn_relu_matmul(
                t, st, g, be,
                params[f"W{i + 1}"], params[f"b{i + 1}"].reshape(1, D), dinv)
        else:
            w2 = jnp.pad(params["hW2"], ((0, 0), (0, D - N_CLASSES)))
            b2 = jnp.pad(params["hb2"], (0, D - N_CLASSES)).reshape(1, D)
            out = _bn_relu_head(
                t, st, g, be,
                params["hW1"], params["hb1"].reshape(1, D), w2, b2)
    return out[:, :N_CLASSES]


# speedup vs baseline: 5.8737x; 1.0342x over previous
import functools

import jax
import jax.numpy as jnp
from jax import lax
from jax.experimental import pallas as pl
from jax.experimental.pallas import tpu as pltpu
from jax.experimental.pallas import tpu_sc as plsc

N_NODES = 10000
D = 128
N_CLASSES = 40
N_LAYERS = 4
EPS = 1e-5

NC = 2
NS = 16
NW = NC * NS
LANES = 16

CHUNK = 128
E_PAD = 327680
TRASH = N_NODES

DEG_NCHUNK = E_PAD // (NW * CHUNK)
DEG_ROWS = 10240
DEG_ZROWS = DEG_ROWS // NS

AGG_NCHUNK = E_PAD // (NS * CHUNK)
ACC_ROWS = 6016
OUT_ROWS = 5120
OUT_SUB = OUT_ROWS // NS
NBUF = 4
BLK = 32
NBLK = AGG_NCHUNK // BLK

ROW_BLK = 2000
GRID = N_NODES // ROW_BLK


def _sc_mesh():
    return plsc.VectorSubcoreMesh(
        core_axis_name="c", subcore_axis_name="s",
        num_cores=NC, num_subcores=NS)


def _sc_degree(dst_r, ones, zeros):
    @functools.partial(
        pl.kernel,
        mesh=_sc_mesh(),
        out_type=jax.ShapeDtypeStruct((NC, DEG_ROWS), jnp.float32),
        scratch_types=[
            pltpu.VMEM((DEG_NCHUNK, CHUNK), jnp.int32),
            pltpu.VMEM((CHUNK,), jnp.float32),
            pltpu.VMEM((DEG_ZROWS,), jnp.float32),
            pltpu.VMEM_SHARED((DEG_ROWS,), jnp.float32),
        ],
    )
    def deg_kernel(dst_hbm, ones_hbm, zeros_hbm, out_hbm, dst_v, ones_v, z_v, acc):
        c = lax.axis_index("c")
        s = lax.axis_index("s")
        wid = s * NC + c
        pltpu.sync_copy(dst_hbm.at[wid], dst_v)
        pltpu.sync_copy(ones_hbm, ones_v)
        pltpu.sync_copy(zeros_hbm, z_v)
        pltpu.sync_copy(z_v, acc.at[pl.ds(s * DEG_ZROWS, DEG_ZROWS)])
        plsc.subcore_barrier()

        @pl.loop(0, DEG_NCHUNK)
        def _(j):
            pltpu.sync_copy(ones_v, acc.at[dst_v.at[j]], add=True)

        plsc.subcore_barrier()
        pltpu.sync_copy(acc.at[pl.ds(s * DEG_ZROWS, DEG_ZROWS)],
                        out_hbm.at[c, pl.ds(s * DEG_ZROWS, DEG_ZROWS)])

    return deg_kernel(dst_r, ones, zeros)


BIN_SLOTS = AGG_NCHUNK * CHUNK + CHUNK
BIN_CH = BIN_SLOTS // CHUNK


def _sc_bin(src_r, dst_r):
    @functools.partial(
        pl.kernel,
        mesh=_sc_mesh(),
        out_type=[
            jax.ShapeDtypeStruct((NC, NS, BIN_SLOTS), jnp.int32),
            jax.ShapeDtypeStruct((NC, NS, BIN_SLOTS), jnp.int32),
            jax.ShapeDtypeStruct((NC, NS, 16), jnp.int32),
        ],
        scratch_types=[
            pltpu.VMEM((AGG_NCHUNK, CHUNK), jnp.int32),
            pltpu.VMEM((AGG_NCHUNK, CHUNK), jnp.int32),
            pltpu.VMEM((BIN_SLOTS,), jnp.int32),
            pltpu.VMEM((BIN_SLOTS,), jnp.int32),
            pltpu.VMEM((16,), jnp.int32),
        ],
    )
    def bin_kernel(src_hbm, dst_hbm, srcb_hbm, dstb_hbm, cnt_hbm,
                   src_v, dst_v, sb, db, cb):
        c = lax.axis_index("c")
        s = lax.axis_index("s")
        pltpu.sync_copy(src_hbm.at[s], src_v)
        pltpu.sync_copy(dst_hbm.at[s], dst_v)
        base = c * OUT_ROWS
        lane = lax.iota(jnp.int32, LANES)

        @pl.loop(0, AGG_NCHUNK, init_carry=jnp.int32(0))
        def compact(j, off):
            for v in range(CHUNK // LANES):
                sl = pl.ds(v * LANES, LANES)
                d = dst_v[j, sl]
                sr = src_v[j, sl]
                dl = d - base
                m = (dl >= 0) & (dl < OUT_ROWS)
                w = jnp.where(m, jnp.int32(1), jnp.int32(0))
                sr_c = sr
                dl_c = dl
                cnt = jnp.int32(0)
                for l in range(LANES):
                    wl = w[l]
                    cond = jnp.where(lane == cnt, wl, jnp.int32(0)) > 0
                    sr_c = jnp.where(cond, sr[l], sr_c)
                    dl_c = jnp.where(cond, dl[l], dl_c)
                    cnt = cnt + wl
                sb[pl.ds(off, LANES)] = sr_c
                db[pl.ds(off, LANES)] = dl_c
                off = off + cnt
            return off

        off = compact
        zs = jnp.zeros((LANES,), jnp.int32)
        for t in range(CHUNK // LANES):
            sb[pl.ds(off + t * LANES, LANES)] = zs
            db[pl.ds(off + t * LANES, LANES)] = OUT_ROWS + lane
        pc = lax.bitwise_and(off + (CHUNK - 1), jnp.int32(-CHUNK))
        cb[...] = jnp.zeros((16,), jnp.int32) + pc
        pltpu.sync_copy(sb, srcb_hbm.at[c, s])
        pltpu.sync_copy(db, dstb_hbm.at[c, s])
        pltpu.sync_copy(cb, cnt_hbm.at[c, s])

    return bin_kernel(src_r, dst_r)


def _sc_aggregate(hp, srcb, dstb, cnt, zrows):
    @functools.partial(
        pl.kernel,
        mesh=_sc_mesh(),
        out_type=jax.ShapeDtypeStruct((NC, OUT_ROWS, D), jnp.float32),
        scratch_types=[
            [pltpu.VMEM((BLK, CHUNK), jnp.int32) for _ in range(2)],
            [pltpu.VMEM((BLK, CHUNK), jnp.int32) for _ in range(2)],
            pltpu.VMEM((16,), jnp.int32),
            [pltpu.VMEM((CHUNK, D), jnp.float32) for _ in range(NBUF)],
            pltpu.VMEM_SHARED((ACC_ROWS, D), jnp.float32),
            [pltpu.SemaphoreType.DMA for _ in range(NBUF)],
            [pltpu.SemaphoreType.DMA for _ in range(NBUF)],
            [pltpu.SemaphoreType.DMA for _ in range(2)],
            [pltpu.SemaphoreType.DMA for _ in range(2)],
        ],
    )
    def agg_kernel(hp_hbm, src_hbm, dst_hbm, cnt_hbm, z_hbm, out_hbm,
                   sidx, didx, cb, rbufs, acc, gsems, ssems, sbsem, dbsem):
        c = lax.axis_index("c")
        s = lax.axis_index("s")
        pltpu.sync_copy(cnt_hbm.at[c, s], cb)
        nch = cb[...][0] // CHUNK

        def bld(k, kb):
            return (
                pltpu.make_async_copy(
                    src_hbm.at[c, s, pl.ds(k * BLK, BLK)], sidx[kb], sbsem[kb]),
                pltpu.make_async_copy(
                    dst_hbm.at[c, s, pl.ds(k * BLK, BLK)], didx[kb], dbsem[kb]),
            )

        @pl.when(nch > 0)
        def _():
            for cp in bld(0, 0):
                cp.start()

        pltpu.sync_copy(z_hbm, rbufs[0])
        pltpu.sync_copy(rbufs[0], acc.at[pl.ds(s * OUT_SUB, CHUNK)])
        pltpu.sync_copy(rbufs[0], acc.at[pl.ds(s * OUT_SUB + CHUNK, CHUNK)])
        pltpu.sync_copy(rbufs[0].at[pl.ds(0, OUT_SUB - 2 * CHUNK)],
                        acc.at[pl.ds(s * OUT_SUB + 2 * CHUNK,
                                     OUT_SUB - 2 * CHUNK)])
        plsc.subcore_barrier()

        def gat(lj, kb, b):
            return pltpu.make_async_copy(
                hp_hbm.at[sidx[kb].at[lj]], rbufs[b], gsems[b])

        def sct(lj, kb, b):
            return pltpu.make_async_copy(
                rbufs[b], acc.at[didx[kb].at[lj]], ssems[b])

        for k in range(NBLK):
            kb = k & 1
            base = k * BLK

            @pl.when(base < nch)
            def _(k=k, kb=kb, base=base):
                for cp in bld(k, kb):
                    cp.wait()
                if k + 1 < NBLK:
                    @pl.when(base + BLK < nch)
                    def _():
                        for cp in bld(k + 1, 1 - kb):
                            cp.start()
                for b in range(NBUF):
                    @pl.when(base + b < nch)
                    def _(b=b):
                        gat(b, kb, b).start()

                @pl.loop(0, BLK // NBUF)
                def _(i, kb=kb, base=base):
                    lj = i * NBUF
                    for b in range(NBUF):
                        @pl.when(base + lj + b < nch)
                        def _(b=b):
                            gat(lj + b, kb, b).wait()
                            sct(lj + b, kb, b).start(add=True)
                    for b in range(NBUF):
                        @pl.when(base + lj + b < nch)
                        def _(b=b):
                            sct(lj + b, kb, b).wait()

                        lj2 = lj + NBUF + b

                        @pl.when((lj2 < BLK) & (base + lj2 < nch))
                        def _(b=b, lj2=lj2):
                            gat(lj2, kb, b).start()

        plsc.subcore_barrier()
        pltpu.sync_copy(acc.at[pl.ds(s * OUT_SUB, OUT_SUB)],
                        out_hbm.at[c, pl.ds(s * OUT_SUB, OUT_SUB)])

    return agg_kernel(hp, srcb, dstb, cnt, zrows)


def _dot(a, w):
    return jnp.dot(a, w, preferred_element_type=jnp.float32,
                   precision=lax.Precision.HIGHEST)


def _first_layer(x, w0, b0, deg0, deg1):
    def body(x_ref, w_ref, b_ref, d0_ref, d1_ref, hp_ref, dinv_ref):
        deg = d0_ref[...] + d1_ref[...] + 1.0
        dinv = lax.rsqrt(jnp.maximum(deg, 1.0))
        dinv_ref[...] = dinv
        hp_ref[...] = (_dot(x_ref[...], w_ref[...]) + b_ref[...]) * dinv

    return pl.pallas_call(
        body,
        grid=(GRID,),
        in_specs=[
            pl.BlockSpec((ROW_BLK, D), lambda i: (i, 0)),
            pl.BlockSpec((D, D), lambda i: (0, 0)),
            pl.BlockSpec((1, D), lambda i: (0, 0)),
            pl.BlockSpec((ROW_BLK, 1), lambda i: (i, 0)),
            pl.BlockSpec((ROW_BLK, 1), lambda i: (i, 0)),
        ],
        out_specs=[
            pl.BlockSpec((ROW_BLK, D), lambda i: (i, 0)),
            pl.BlockSpec((ROW_BLK, 1), lambda i: (i, 0)),
        ],
        out_shape=[
            jax.ShapeDtypeStruct((N_NODES, D), jnp.float32),
            jax.ShapeDtypeStruct((N_NODES, 1), jnp.float32),
        ],
    )(x, w0, b0, deg0, deg1)


def _combine_stats(aggf, hp, dinv):
    def body(a_ref, hp_ref, dinv_ref, t_ref, st_ref):
        t = (a_ref[...] + hp_ref[...]) * dinv_ref[...]
        t_ref[...] = t
        s0 = jnp.sum(t, axis=0, keepdims=True)
        s1 = jnp.sum(t * t, axis=0, keepdims=True)
        blk = jnp.concatenate([s0, s1, jnp.zeros((6, D), jnp.float32)], axis=0)

        @pl.when(pl.program_id(0) == 0)
        def _():
            st_ref[...] = blk

        @pl.when(pl.program_id(0) != 0)
        def _():
            st_ref[...] += blk

    return pl.pallas_call(
        body,
        grid=(GRID,),
        in_specs=[
            pl.BlockSpec((ROW_BLK, D), lambda i: (i, 0)),
            pl.BlockSpec((ROW_BLK, D), lambda i: (i, 0)),
            pl.BlockSpec((ROW_BLK, 1), lambda i: (i, 0)),
        ],
        out_specs=[
            pl.BlockSpec((ROW_BLK, D), lambda i: (i, 0)),
            pl.BlockSpec((8, D), lambda i: (0, 0)),
        ],
        out_shape=[
            jax.ShapeDtypeStruct((N_NODES, D), jnp.float32),
            jax.ShapeDtypeStruct((8, D), jnp.float32),
        ],
    )(aggf, hp, dinv)


def _bn_relu_matmul(t, st, g, be, wn, bn_, dinv):
    def body(t_ref, st_ref, g_ref, be_ref, w_ref, b_ref, dinv_ref, hp_ref):
        n = jnp.float32(N_NODES)
        mean = st_ref[0:1, :] / n
        var = st_ref[1:2, :] / n - mean * mean
        inv = lax.rsqrt(var + EPS)
        xb = jax.nn.relu((t_ref[...] - mean) * inv * g_ref[...] + be_ref[...])
        hp_ref[...] = (_dot(xb, w_ref[...]) + b_ref[...]) * dinv_ref[...]

    return pl.pallas_call(
        body,
        grid=(GRID,),
        in_specs=[
            pl.BlockSpec((ROW_BLK, D), lambda i: (i, 0)),
            pl.BlockSpec((8, D), lambda i: (0, 0)),
            pl.BlockSpec((1, D), lambda i: (0, 0)),
            pl.BlockSpec((1, D), lambda i: (0, 0)),
            pl.BlockSpec((D, D), lambda i: (0, 0)),
            pl.BlockSpec((1, D), lambda i: (0, 0)),
            pl.BlockSpec((ROW_BLK, 1), lambda i: (i, 0)),
        ],
        out_specs=pl.BlockSpec((ROW_BLK, D), lambda i: (i, 0)),
        out_shape=jax.ShapeDtypeStruct((N_NODES, D), jnp.float32),
    )(t, st, g, be, wn, bn_, dinv)


def _bn_relu_head(t, st, g, be, w1, b1, w2, b2):
    def body(t_ref, st_ref, g_ref, be_ref, w1_ref, b1_ref, w2_ref, b2_ref, o_ref):
        n = jnp.float32(N_NODES)
        mean = st_ref[0:1, :] / n
        var = st_ref[1:2, :] / n - mean * mean
        inv = lax.rsqrt(var + EPS)
        xb = jax.nn.relu((t_ref[...] - mean) * inv * g_ref[...] + be_ref[...])
        h1 = jax.nn.relu(_dot(xb, w1_ref[...]) + b1_ref[...])
        o_ref[...] = _dot(h1, w2_ref[...]) + b2_ref[...]

    return pl.pallas_call(
        body,
        grid=(GRID,),
        in_specs=[
            pl.BlockSpec((ROW_BLK, D), lambda i: (i, 0)),
            pl.BlockSpec((8, D), lambda i: (0, 0)),
            pl.BlockSpec((1, D), lambda i: (0, 0)),
            pl.BlockSpec((1, D), lambda i: (0, 0)),
            pl.BlockSpec((D, D), lambda i: (0, 0)),
            pl.BlockSpec((1, D), lambda i: (0, 0)),
            pl.BlockSpec((D, D), lambda i: (0, 0)),
            pl.BlockSpec((1, D), lambda i: (0, 0)),
        ],
        out_specs=pl.BlockSpec((ROW_BLK, D), lambda i: (i, 0)),
        out_shape=jax.ShapeDtypeStruct((N_NODES, D), jnp.float32),
    )(t, st, g, be, w1, b1, w2, b2)


def kernel(x, edge_index, batch, params):
    del batch

    src = edge_index[0].astype(jnp.int32)
    dst = edge_index[1].astype(jnp.int32)
    e = src.shape[0]
    pad = E_PAD - e
    src_f = jnp.concatenate([src, jnp.zeros((pad,), jnp.int32)])
    dst_f = jnp.concatenate([dst, jnp.full((pad,), TRASH, jnp.int32)])
    src_a = src_f.reshape(NS, AGG_NCHUNK, CHUNK)
    dst_a = dst_f.reshape(NS, AGG_NCHUNK, CHUNK)
    dst_d = dst_f.reshape(NW, DEG_NCHUNK, CHUNK)

    ones = jnp.ones((CHUNK,), jnp.float32)
    zeros = jnp.zeros((DEG_ZROWS,), jnp.float32)
    zrows = jnp.zeros((CHUNK, D), jnp.float32)

    srcb, dstb, cntb = _sc_bin(src_a, dst_a)
    srcb4 = srcb.reshape(NC, NS, BIN_CH, CHUNK)
    dstb4 = dstb.reshape(NC, NS, BIN_CH, CHUNK)

    deg_p = _sc_degree(dst_d, ones, zeros)
    deg0 = deg_p[0, :N_NODES].reshape(N_NODES, 1)
    deg1 = deg_p[1, :N_NODES].reshape(N_NODES, 1)

    hp, dinv = _first_layer(
        x, params["W0"], params["b0"].reshape(1, D), deg0, deg1)

    for i in range(N_LAYERS):
        agg = _sc_aggregate(hp, srcb4, dstb4, cntb, zrows)
        aggf = agg.reshape(NC * OUT_ROWS, D)
        t, st = _combine_stats(aggf, hp, dinv)
        g = params[f"g{i}"].reshape(1, D)
        be = params[f"be{i}"].reshape(1, D)
        if i + 1 < N_LAYERS:
            hp = _bn_relu_matmul(
                t, st, g, be,
                params[f"W{i + 1}"], params[f"b{i + 1}"].reshape(1, D), dinv)
        else:
            w2 = jnp.pad(params["hW2"], ((0, 0), (0, D - N_CLASSES)))
            b2 = jnp.pad(params["hb2"], (0, D - N_CLASSES)).reshape(1, D)
            out = _bn_relu_head(
                t, st, g, be,
                params["hW1"], params["hb1"].reshape(1, D), w2, b2)
    return out[:, :N_CLASSES]


# speedup vs baseline: 15.0021x; 2.5541x over previous
import functools

import jax
import jax.numpy as jnp
from jax import lax
from jax.experimental import pallas as pl
from jax.experimental.pallas import tpu as pltpu
from jax.experimental.pallas import tpu_sc as plsc

N_NODES = 10000
D = 128
N_CLASSES = 40
N_LAYERS = 4
EPS = 1e-5

NC = 2
NS = 16
NW = NC * NS
LANES = 16

CHUNK = 128
E_PAD = 327680
TRASH = N_NODES

DEG_NCHUNK = E_PAD // (NW * CHUNK)
DEG_ROWS = 10240
DEG_ZROWS = DEG_ROWS // NS

AGG_NCHUNK = E_PAD // (NS * CHUNK)
ACC_ROWS = 6016
OUT_ROWS = 5120
OUT_SUB = OUT_ROWS // NS
NBUF = 4
BLK = 32
NBLK = AGG_NCHUNK // BLK

ROW_BLK = 2000
GRID = N_NODES // ROW_BLK


def _sc_mesh():
    return plsc.VectorSubcoreMesh(
        core_axis_name="c", subcore_axis_name="s",
        num_cores=NC, num_subcores=NS)


def _sc_degree(dst_r, ones, zeros):
    @functools.partial(
        pl.kernel,
        mesh=_sc_mesh(),
        out_type=jax.ShapeDtypeStruct((NC, DEG_ROWS), jnp.float32),
        scratch_types=[
            pltpu.VMEM((DEG_NCHUNK, CHUNK), jnp.int32),
            pltpu.VMEM((CHUNK,), jnp.float32),
            pltpu.VMEM((DEG_ZROWS,), jnp.float32),
            pltpu.VMEM_SHARED((DEG_ROWS,), jnp.float32),
        ],
    )
    def deg_kernel(dst_hbm, ones_hbm, zeros_hbm, out_hbm, dst_v, ones_v, z_v, acc):
        c = lax.axis_index("c")
        s = lax.axis_index("s")
        wid = s * NC + c
        pltpu.sync_copy(dst_hbm.at[wid], dst_v)
        pltpu.sync_copy(ones_hbm, ones_v)
        pltpu.sync_copy(zeros_hbm, z_v)
        pltpu.sync_copy(z_v, acc.at[pl.ds(s * DEG_ZROWS, DEG_ZROWS)])
        plsc.subcore_barrier()

        @pl.loop(0, DEG_NCHUNK)
        def _(j):
            pltpu.sync_copy(ones_v, acc.at[dst_v.at[j]], add=True)

        plsc.subcore_barrier()
        pltpu.sync_copy(acc.at[pl.ds(s * DEG_ZROWS, DEG_ZROWS)],
                        out_hbm.at[c, pl.ds(s * DEG_ZROWS, DEG_ZROWS)])

    return deg_kernel(dst_r, ones, zeros)


BIN_SLOTS = AGG_NCHUNK * CHUNK + CHUNK
BIN_CH = BIN_SLOTS // CHUNK


def _sc_bin(src_r, dst_r):
    @functools.partial(
        pl.kernel,
        mesh=_sc_mesh(),
        out_type=[
            jax.ShapeDtypeStruct((NC, NS, BIN_SLOTS), jnp.int32),
            jax.ShapeDtypeStruct((NC, NS, BIN_SLOTS), jnp.int32),
            jax.ShapeDtypeStruct((NC, NS, 16), jnp.int32),
        ],
        scratch_types=[
            pltpu.VMEM((AGG_NCHUNK, CHUNK), jnp.int32),
            pltpu.VMEM((AGG_NCHUNK, CHUNK), jnp.int32),
            pltpu.VMEM((BIN_SLOTS,), jnp.int32),
            pltpu.VMEM((BIN_SLOTS,), jnp.int32),
            pltpu.VMEM((16,), jnp.int32),
        ],
    )
    def bin_kernel(src_hbm, dst_hbm, srcb_hbm, dstb_hbm, cnt_hbm,
                   src_v, dst_v, sb, db, cb):
        c = lax.axis_index("c")
        s = lax.axis_index("s")
        pltpu.sync_copy(src_hbm.at[s], src_v)
        pltpu.sync_copy(dst_hbm.at[s], dst_v)
        base = c * OUT_ROWS
        lane = lax.iota(jnp.int32, LANES)

        @pl.loop(0, AGG_NCHUNK, init_carry=jnp.int32(0))
        def compact(j, off):
            for v in range(CHUNK // LANES):
                sl = pl.ds(v * LANES, LANES)
                d = dst_v[j, sl]
                sr = src_v[j, sl]
                dl = d - base
                m = (dl >= 0) & (dl < OUT_ROWS)
                w = jnp.where(m, jnp.int32(1), jnp.int32(0))
                sr_c = sr
                dl_c = dl
                cnt = jnp.int32(0)
                for l in range(LANES):
                    wl = w[l]
                    cond = jnp.where(lane == cnt, wl, jnp.int32(0)) > 0
                    sr_c = jnp.where(cond, sr[l], sr_c)
                    dl_c = jnp.where(cond, dl[l], dl_c)
                    cnt = cnt + wl
                sb[pl.ds(off, LANES)] = sr_c
                db[pl.ds(off, LANES)] = dl_c
                off = off + cnt
            return off

        off = compact
        zs = jnp.zeros((LANES,), jnp.int32)
        for t in range(CHUNK // LANES):
            sb[pl.ds(off + t * LANES, LANES)] = zs
            db[pl.ds(off + t * LANES, LANES)] = OUT_ROWS + lane
        pc = lax.bitwise_and(off + (CHUNK - 1), jnp.int32(-CHUNK))
        cb[...] = jnp.zeros((16,), jnp.int32) + pc
        pltpu.sync_copy(sb, srcb_hbm.at[c, s])
        pltpu.sync_copy(db, dstb_hbm.at[c, s])
        pltpu.sync_copy(cb, cnt_hbm.at[c, s])

    return bin_kernel(src_r, dst_r)


def _sc_aggregate(hp, srcb, dstb, cnt, zrows):
    @functools.partial(
        pl.kernel,
        mesh=_sc_mesh(),
        out_type=jax.ShapeDtypeStruct((NC, OUT_ROWS, D), jnp.float32),
        scratch_types=[
            [pltpu.VMEM((BLK, CHUNK), jnp.int32) for _ in range(2)],
            [pltpu.VMEM((BLK, CHUNK), jnp.int32) for _ in range(2)],
            pltpu.VMEM((16,), jnp.int32),
            [pltpu.VMEM((CHUNK, D), jnp.float32) for _ in range(NBUF)],
            pltpu.VMEM_SHARED((ACC_ROWS, D), jnp.float32),
            [pltpu.SemaphoreType.DMA for _ in range(NBUF)],
            [pltpu.SemaphoreType.DMA for _ in range(NBUF)],
            [pltpu.SemaphoreType.DMA for _ in range(2)],
            [pltpu.SemaphoreType.DMA for _ in range(2)],
        ],
    )
    def agg_kernel(hp_hbm, src_hbm, dst_hbm, cnt_hbm, z_hbm, out_hbm,
                   sidx, didx, cb, rbufs, acc, gsems, ssems, sbsem, dbsem):
        c = lax.axis_index("c")
        s = lax.axis_index("s")
        pltpu.sync_copy(cnt_hbm.at[c, s], cb)
        nch = cb[...][0] // CHUNK

        def bld(k, kb):
            return (
                pltpu.make_async_copy(
                    src_hbm.at[c, s, pl.ds(k * BLK, BLK)], sidx[kb], sbsem[kb]),
                pltpu.make_async_copy(
                    dst_hbm.at[c, s, pl.ds(k * BLK, BLK)], didx[kb], dbsem[kb]),
            )

        @pl.when(nch > 0)
        def _():
            for cp in bld(0, 0):
                cp.start()

        pltpu.sync_copy(z_hbm, rbufs[0])
        pltpu.sync_copy(rbufs[0], acc.at[pl.ds(s * OUT_SUB, CHUNK)])
        pltpu.sync_copy(rbufs[0], acc.at[pl.ds(s * OUT_SUB + CHUNK, CHUNK)])
        pltpu.sync_copy(rbufs[0].at[pl.ds(0, OUT_SUB - 2 * CHUNK)],
                        acc.at[pl.ds(s * OUT_SUB + 2 * CHUNK,
                                     OUT_SUB - 2 * CHUNK)])
        plsc.subcore_barrier()

        def gat(lj, kb, b):
            return pltpu.make_async_copy(
                hp_hbm.at[sidx[kb].at[lj]], rbufs[b], gsems[b])

        def sct(lj, kb, b):
            return pltpu.make_async_copy(
                rbufs[b], acc.at[didx[kb].at[lj]], ssems[b])

        for k in range(NBLK):
            kb = k & 1
            base = k * BLK

            @pl.when(base < nch)
            def _(k=k, kb=kb, base=base):
                for cp in bld(k, kb):
                    cp.wait()
                if k + 1 < NBLK:
                    @pl.when(base + BLK < nch)
                    def _():
                        for cp in bld(k + 1, 1 - kb):
                            cp.start()
                for b in range(NBUF):
                    @pl.when(base + b < nch)
                    def _(b=b):
                        gat(b, kb, b).start()

                @pl.loop(0, BLK // NBUF)
                def _(i, kb=kb, base=base):
                    lj = i * NBUF
                    for b in range(NBUF):
                        @pl.when(base + lj + b < nch)
                        def _(b=b):
                            gat(lj + b, kb, b).wait()
                            sct(lj + b, kb, b).start(add=True)
                    for b in range(NBUF):
                        @pl.when(base + lj + b < nch)
                        def _(b=b):
                            sct(lj + b, kb, b).wait()

                        lj2 = lj + NBUF + b

                        @pl.when((lj2 < BLK) & (base + lj2 < nch))
                        def _(b=b, lj2=lj2):
                            gat(lj2, kb, b).start()

        plsc.subcore_barrier()
        pltpu.sync_copy(acc.at[pl.ds(s * OUT_SUB, OUT_SUB)],
                        out_hbm.at[c, pl.ds(s * OUT_SUB, OUT_SUB)])

    return agg_kernel(hp, srcb, dstb, cnt, zrows)


def _dot(a, w):
    return jnp.dot(a, w, preferred_element_type=jnp.float32,
                   precision=lax.Precision.HIGHEST)


def _first_layer(x, w0, b0, deg0, deg1):
    def body(x_ref, w_ref, b_ref, d0_ref, d1_ref, hp_ref, dinv_ref):
        deg = d0_ref[...] + d1_ref[...] + 1.0
        dinv = lax.rsqrt(jnp.maximum(deg, 1.0))
        dinv_ref[...] = dinv
        hp_ref[...] = (_dot(x_ref[...], w_ref[...]) + b_ref[...]) * dinv

    return pl.pallas_call(
        body,
        grid=(GRID,),
        in_specs=[
            pl.BlockSpec((ROW_BLK, D), lambda i: (i, 0)),
            pl.BlockSpec((D, D), lambda i: (0, 0)),
            pl.BlockSpec((1, D), lambda i: (0, 0)),
            pl.BlockSpec((ROW_BLK, 1), lambda i: (i, 0)),
            pl.BlockSpec((ROW_BLK, 1), lambda i: (i, 0)),
        ],
        out_specs=[
            pl.BlockSpec((ROW_BLK, D), lambda i: (i, 0)),
            pl.BlockSpec((ROW_BLK, 1), lambda i: (i, 0)),
        ],
        out_shape=[
            jax.ShapeDtypeStruct((N_NODES, D), jnp.float32),
            jax.ShapeDtypeStruct((N_NODES, 1), jnp.float32),
        ],
    )(x, w0, b0, deg0, deg1)


def _combine_stats(aggf, hp, dinv):
    def body(a_ref, hp_ref, dinv_ref, t_ref, st_ref):
        t = (a_ref[...] + hp_ref[...]) * dinv_ref[...]
        t_ref[...] = t
        s0 = jnp.sum(t, axis=0, keepdims=True)
        s1 = jnp.sum(t * t, axis=0, keepdims=True)
        blk = jnp.concatenate([s0, s1, jnp.zeros((6, D), jnp.float32)], axis=0)

        @pl.when(pl.program_id(0) == 0)
        def _():
            st_ref[...] = blk

        @pl.when(pl.program_id(0) != 0)
        def _():
            st_ref[...] += blk

    return pl.pallas_call(
        body,
        grid=(GRID,),
        in_specs=[
            pl.BlockSpec((ROW_BLK, D), lambda i: (i, 0)),
            pl.BlockSpec((ROW_BLK, D), lambda i: (i, 0)),
            pl.BlockSpec((ROW_BLK, 1), lambda i: (i, 0)),
        ],
        out_specs=[
            pl.BlockSpec((ROW_BLK, D), lambda i: (i, 0)),
            pl.BlockSpec((8, D), lambda i: (0, 0)),
        ],
        out_shape=[
            jax.ShapeDtypeStruct((N_NODES, D), jnp.float32),
            jax.ShapeDtypeStruct((8, D), jnp.float32),
        ],
    )(aggf, hp, dinv)


def _bn_relu_matmul(t, st, g, be, wn, bn_, dinv):
    def body(t_ref, st_ref, g_ref, be_ref, w_ref, b_ref, dinv_ref, hp_ref):
        n = jnp.float32(N_NODES)
        mean = st_ref[0:1, :] / n
        var = st_ref[1:2, :] / n - mean * mean
        inv = lax.rsqrt(var + EPS)
        xb = jax.nn.relu((t_ref[...] - mean) * inv * g_ref[...] + be_ref[...])
        hp_ref[...] = (_dot(xb, w_ref[...]) + b_ref[...]) * dinv_ref[...]

    return pl.pallas_call(
        body,
        grid=(GRID,),
        in_specs=[
            pl.BlockSpec((ROW_BLK, D), lambda i: (i, 0)),
            pl.BlockSpec((8, D), lambda i: (0, 0)),
            pl.BlockSpec((1, D), lambda i: (0, 0)),
            pl.BlockSpec((1, D), lambda i: (0, 0)),
            pl.BlockSpec((D, D), lambda i: (0, 0)),
            pl.BlockSpec((1, D), lambda i: (0, 0)),
            pl.BlockSpec((ROW_BLK, 1), lambda i: (i, 0)),
        ],
        out_specs=pl.BlockSpec((ROW_BLK, D), lambda i: (i, 0)),
        out_shape=jax.ShapeDtypeStruct((N_NODES, D), jnp.float32),
    )(t, st, g, be, wn, bn_, dinv)


def _bn_relu_head(t, st, g, be, w1, b1, w2, b2):
    def body(t_ref, st_ref, g_ref, be_ref, w1_ref, b1_ref, w2_ref, b2_ref, o_ref):
        n = jnp.float32(N_NODES)
        mean = st_ref[0:1, :] / n
        var = st_ref[1:2, :] / n - mean * mean
        inv = lax.rsqrt(var + EPS)
        xb = jax.nn.relu((t_ref[...] - mean) * inv * g_ref[...] + be_ref[...])
        h1 = jax.nn.relu(_dot(xb, w1_ref[...]) + b1_ref[...])
        o_ref[...] = _dot(h1, w2_ref[...]) + b2_ref[...]

    return pl.pallas_call(
        body,
        grid=(GRID,),
        in_specs=[
            pl.BlockSpec((ROW_BLK, D), lambda i: (i, 0)),
            pl.BlockSpec((8, D), lambda i: (0, 0)),
            pl.BlockSpec((1, D), lambda i: (0, 0)),
            pl.BlockSpec((1, D), lambda i: (0, 0)),
            pl.BlockSpec((D, D), lambda i: (0, 0)),
            pl.BlockSpec((1, D), lambda i: (0, 0)),
            pl.BlockSpec((D, D), lambda i: (0, 0)),
            pl.BlockSpec((1, D), lambda i: (0, 0)),
        ],
        out_specs=pl.BlockSpec((ROW_BLK, D), lambda i: (i, 0)),
        out_shape=jax.ShapeDtypeStruct((N_NODES, D), jnp.float32),
    )(t, st, g, be, w1, b1, w2, b2)


def kernel(x, edge_index, batch, params):
    del batch

    src = edge_index[0].astype(jnp.int32)
    dst = edge_index[1].astype(jnp.int32)
    e = src.shape[0]
    pad = E_PAD - e
    pad_ids = jnp.arange(pad, dtype=jnp.int32)
    src_f = jnp.concatenate([src, pad_ids % N_NODES])
    dst_f = jnp.concatenate([dst, TRASH + pad_ids % (NC * OUT_ROWS - N_NODES)])
    src_a = src_f.reshape(NS, AGG_NCHUNK, CHUNK)
    dst_a = dst_f.reshape(NS, AGG_NCHUNK, CHUNK)
    dst_d = dst_f.reshape(NW, DEG_NCHUNK, CHUNK)

    ones = jnp.ones((CHUNK,), jnp.float32)
    zeros = jnp.zeros((DEG_ZROWS,), jnp.float32)
    zrows = jnp.zeros((CHUNK, D), jnp.float32)

    srcb, dstb, cntb = _sc_bin(src_a, dst_a)
    srcb4 = srcb.reshape(NC, NS, BIN_CH, CHUNK)
    dstb4 = dstb.reshape(NC, NS, BIN_CH, CHUNK)

    deg_p = _sc_degree(dst_d, ones, zeros)
    deg0 = deg_p[0, :N_NODES].reshape(N_NODES, 1)
    deg1 = deg_p[1, :N_NODES].reshape(N_NODES, 1)

    hp, dinv = _first_layer(
        x, params["W0"], params["b0"].reshape(1, D), deg0, deg1)

    for i in range(N_LAYERS):
        agg = _sc_aggregate(hp, srcb4, dstb4, cntb, zrows)
        aggf = agg.reshape(NC * OUT_ROWS, D)
        t, st = _combine_stats(aggf, hp, dinv)
        g = params[f"g{i}"].reshape(1, D)
        be = params[f"be{i}"].reshape(1, D)
        if i + 1 < N_LAYERS:
            hp = _bn_relu_matmul(
                t, st, g, be,
                params[f"W{i + 1}"], params[f"b{i + 1}"].reshape(1, D), dinv)
        else:
            w2 = jnp.pad(params["hW2"], ((0, 0), (0, D - N_CLASSES)))
            b2 = jnp.pad(params["hb2"], (0, D - N_CLASSES)).reshape(1, D)
            out = _bn_relu_head(
                t, st, g, be,
                params["hW1"], params["hb1"].reshape(1, D), w2, b2)
    return out[:, :N_CLASSES]
